# baseline (speedup 1.0000x reference)
import jax
import jax.numpy as jnp
from jax.experimental import pallas as pl

NUM_EXPERTS = 64
TOP_K = 8
HIDDEN = 4096
TOKENS = 16384
BLOCK_T = 1024


def _router_block(x_ref, w_ref, scores_ref, idx_ref, logits_ref):
    x = x_ref[...]
    w = w_ref[...]
    logits_t = jax.lax.dot_general(
        w, x, (((1,), (1,)), ((), ())), preferred_element_type=jnp.float32
    )
    logits_ref[...] = logits_t.T

    m = jnp.max(logits_t, axis=0, keepdims=True)
    s = jnp.sum(jnp.exp(logits_t - m), axis=0, keepdims=True)

    eidx = jax.lax.broadcasted_iota(jnp.int32, logits_t.shape, 0).astype(
        jnp.float32
    )
    work = logits_t
    cms = []
    cis = []
    for _ in range(TOP_K):
        cm = jnp.max(work, axis=0, keepdims=True)
        ci = jnp.min(
            jnp.where(work == cm, eidx, float(NUM_EXPERTS)), axis=0, keepdims=True
        )
        cms.append(cm)
        cis.append(ci)
        work = jnp.where(eidx == ci, -jnp.inf, work)
    cms8 = jnp.concatenate(cms, axis=0)
    cis8 = jnp.concatenate(cis, axis=0)
    scores_ref[...] = (jnp.exp(cms8 - m) / s).T
    idx_ref[...] = cis8.T.astype(jnp.int32)


@jax.jit
def kernel(input, weight):
    grid = (TOKENS // BLOCK_T,)
    return pl.pallas_call(
        _router_block,
        grid=grid,
        in_specs=[
            pl.BlockSpec((BLOCK_T, HIDDEN), lambda i: (i, 0)),
            pl.BlockSpec((NUM_EXPERTS, HIDDEN), lambda i: (0, 0)),
        ],
        out_specs=[
            pl.BlockSpec((BLOCK_T, TOP_K), lambda i: (i, 0)),
            pl.BlockSpec((BLOCK_T, TOP_K), lambda i: (i, 0)),
            pl.BlockSpec((BLOCK_T, NUM_EXPERTS), lambda i: (i, 0)),
        ],
        out_shape=[
            jax.ShapeDtypeStruct((TOKENS, TOP_K), jnp.float32),
            jax.ShapeDtypeStruct((TOKENS, TOP_K), jnp.int32),
            jax.ShapeDtypeStruct((TOKENS, NUM_EXPERTS), jnp.float32),
        ],
    )(input, weight)
